# baseline (speedup 1.0000x reference)
import functools

import jax
import jax.numpy as jnp
from jax import lax
from jax.experimental import pallas as pl
from jax.experimental.pallas import tpu as pltpu
from jax.experimental.pallas import tpu_sc as plsc

_NUM_ITEM = 100000
_B = 1024
_L = 20
_H = 64

_NC = 1
_NS = 16
_NW = _NC * _NS
_RPW = _B // _NW
_TR = _RPW // 8
_SPW = _RPW * _L
_G = 8 * _L // 16
_W = 6400
_KFULL = _NUM_ITEM // _W
_WLAST = _NUM_ITEM - _KFULL * _W


def _tc_probs_body(am_ref, lm_ref, seq_ref, wr_ref, ur_ref, vw_ref, vb_ref,
                   vals_ref, rows_ref, cols_ref):
    a = jnp.dot(
        am_ref[...].reshape(_B * _L, _H),
        ur_ref[...].T,
        preferred_element_type=jnp.float32,
        precision=lax.Precision.HIGHEST,
    )
    m = jnp.dot(
        lm_ref[...],
        wr_ref[...].T,
        preferred_element_type=jnp.float32,
        precision=lax.Precision.HIGHEST,
    )
    t = jnp.tanh(a.reshape(_B, _L, _H) + m[:, None, :])
    s = jnp.sum(t * vw_ref[...][None], axis=-1) + vb_ref[...]
    smax = jnp.max(s, axis=1, keepdims=True)
    e = jnp.exp(s - smax)
    p = e / jnp.sum(e, axis=1, keepdims=True)
    seq = seq_ref[...]
    acc = jnp.zeros_like(p)
    for l in range(_L):
        acc = acc + jnp.where(seq == seq[:, l:l + 1], p[:, l:l + 1], 0.0)
    vals_ref[...] = acc
    rows_ref[...] = lax.broadcasted_iota(jnp.int32, (_B, _L), 0) % 8
    cols_ref[...] = seq


def _tc_probs(all_memory, last_memory, seq, Wr, Ur, Vr_w, Vr_b):
    return pl.pallas_call(
        _tc_probs_body,
        out_shape=[
            jax.ShapeDtypeStruct((_B, _L), jnp.float32),
            jax.ShapeDtypeStruct((_B, _L), jnp.int32),
            jax.ShapeDtypeStruct((_B, _L), jnp.int32),
        ],
    )(all_memory, last_memory, seq, Wr, Ur, Vr_w, Vr_b)


def _sc_scatter_body(rows_hbm, cols_hbm, vals_hbm, zeros_hbm, zeros_tail_hbm,
                     out_hbm, rows_v, cols_v, vals_v, buf, tail_buf,
                     dma_sem, stage_sem):
    wid = lax.axis_index("s") * _NC + lax.axis_index("c")
    stage_r = pltpu.async_copy(
        rows_hbm.at[pl.ds(wid * _SPW, _SPW)], rows_v, stage_sem)
    stage_c = pltpu.async_copy(
        cols_hbm.at[pl.ds(wid * _SPW, _SPW)], cols_v, stage_sem)
    stage_v = pltpu.async_copy(
        vals_hbm.at[pl.ds(wid * _SPW, _SPW)], vals_v, stage_sem)
    stage_z = pltpu.async_copy(zeros_hbm, buf, stage_sem)
    stage_zt = pltpu.async_copy(zeros_tail_hbm, tail_buf, stage_sem)
    stage_r.wait()
    stage_c.wait()
    stage_v.wait()
    stage_z.wait()
    stage_zt.wait()

    def chunk_scatter(target, t, col_base, width, write_vals):
        for j in range(_G):
            o = (t * _G + j) * 16
            r16 = rows_v[pl.ds(o, 16)]
            c16 = cols_v[pl.ds(o, 16)] - col_base
            mask = (c16 >= 0) & (c16 < width)
            c16 = jnp.where(mask, c16, 0)
            x = vals_v[pl.ds(o, 16)] if write_vals else jnp.zeros(
                (16,), jnp.float32)
            plsc.store_scatter(target, [r16, c16], x, mask=mask)

    for t in range(_TR):
        row0 = pl.multiple_of((wid * _TR + t) * 8, 8)

        def body(k, _, t=t, row0=row0):
            col_base = pl.multiple_of(k * _W, 128)
            chunk_scatter(buf, t, col_base, _W, True)
            cp = pltpu.async_copy(
                buf,
                out_hbm.at[pl.ds(row0, 8), pl.ds(col_base, _W)],
                dma_sem,
            )
            cp.wait()
            chunk_scatter(buf, t, col_base, _W, False)
            return 0

        lax.fori_loop(0, _KFULL, body, 0)
        col_base = _KFULL * _W
        chunk_scatter(tail_buf, t, col_base, _WLAST, True)
        cp = pltpu.async_copy(
            tail_buf,
            out_hbm.at[pl.ds(row0, 8), pl.ds(col_base, _WLAST)],
            dma_sem,
        )
        cp.wait()
        chunk_scatter(tail_buf, t, col_base, _WLAST, False)


@functools.lru_cache(maxsize=1)
def _make_sc_scatter():
    return pl.kernel(
        _sc_scatter_body,
        out_type=jax.ShapeDtypeStruct((_B, _NUM_ITEM), jnp.float32),
        mesh=plsc.VectorSubcoreMesh(core_axis_name="c", subcore_axis_name="s", num_cores=1),
        compiler_params=pltpu.CompilerParams(
            needs_layout_passes=False, use_tc_tiling_on_sc=True),
        scratch_types=[
            pltpu.VMEM((_SPW,), jnp.int32),
            pltpu.VMEM((_SPW,), jnp.int32),
            pltpu.VMEM((_SPW,), jnp.float32),
            pltpu.VMEM((8, _W), jnp.float32),
            pltpu.VMEM((8, _WLAST), jnp.float32),
            pltpu.SemaphoreType.DMA,
            pltpu.SemaphoreType.DMA,
        ],
    )


def kernel(all_memory, last_memory, seq_item, Wr, Ur, Vr_w, Vr_b):
    seq = seq_item.astype(jnp.int32)
    vals, rows, cols = _tc_probs(all_memory, last_memory, seq, Wr, Ur,
                                 Vr_w, Vr_b)
    zeros = jnp.zeros((8, _W), jnp.float32)
    zeros_tail = jnp.zeros((8, _WLAST), jnp.float32)
    return _make_sc_scatter()(
        rows.reshape(-1), cols.reshape(-1), vals.reshape(-1), zeros,
        zeros_tail)


# speedup vs baseline: 4.1100x; 4.1100x over previous
import functools

import jax
import jax.numpy as jnp
from jax import lax
from jax.experimental import pallas as pl
from jax.experimental.pallas import tpu as pltpu
from jax.experimental.pallas import tpu_sc as plsc

_NUM_ITEM = 100000
_B = 1024
_L = 20
_H = 64

_IST = 4
_BST = 8
_IPS = _NUM_ITEM // _IST
_SPS = 128 * _L
_NG = _SPS // 16
_T = 448
_KF = 55
_TAIL = _IPS - _KF * _T


def _tc_probs_body(am_ref, lm_ref, seq_ref, wr_ref, ur_ref, vw_ref, vb_ref,
                   vals_ref, items_ref):
    m = jnp.dot(wr_ref[...], lm_ref[...],
                preferred_element_type=jnp.float32,
                precision=lax.Precision.HIGHEST)
    vw = vw_ref[...]
    rows = []
    for l in range(_L):
        z = jnp.tanh(jnp.dot(ur_ref[...], am_ref[l],
                             preferred_element_type=jnp.float32,
                             precision=lax.Precision.HIGHEST) + m)
        rows.append(jnp.dot(vw, z, preferred_element_type=jnp.float32,
                            precision=lax.Precision.HIGHEST))
    s = jnp.concatenate(rows, axis=0) + vb_ref[...]
    smax = jnp.max(s, axis=0, keepdims=True)
    e = jnp.exp(s - smax)
    p = e / jnp.sum(e, axis=0, keepdims=True)
    seq = seq_ref[...]
    acc = jnp.zeros_like(p)
    for l in range(_L):
        acc = acc + jnp.where(seq == seq[l:l + 1, :], p[l:l + 1, :], 0.0)
    vals_ref[...] = acc
    items_ref[...] = seq


def _tc_probs(all_memory, last_memory, seq, Wr, Ur, Vr_w, Vr_b):
    return pl.pallas_call(
        _tc_probs_body,
        out_shape=[
            jax.ShapeDtypeStruct((_L, _B), jnp.float32),
            jax.ShapeDtypeStruct((_L, _B), jnp.int32),
        ],
    )(all_memory, last_memory, seq, Wr, Ur, Vr_w, Vr_b)


_NEG = -(1 << 30)


def _sc_scatter_body(items_hbm, vals_hbm, zeros_hbm, out_hbm,
                     items_v, vals_v, items_c, cols_c, vals_c,
                     buf_a, buf_b, sem_a, sem_b, stage_sem):
    wid = lax.axis_index("s") * 2 + lax.axis_index("c")
    jb = wid // _IST
    it = wid % _IST
    stage_i = pltpu.async_copy(
        items_hbm.at[:, pl.ds(jb * 128, 128)], items_v, stage_sem)
    stage_v = pltpu.async_copy(
        vals_hbm.at[:, pl.ds(jb * 128, 128)], vals_v, stage_sem)
    stage_za = pltpu.async_copy(zeros_hbm, buf_a, stage_sem)
    stage_zb = pltpu.async_copy(zeros_hbm, buf_b, stage_sem)
    stage_i.wait()
    stage_v.wait()
    stage_za.wait()
    stage_zb.wait()

    item0 = it * _IPS
    col0 = pl.multiple_of(jb * 128, 128)

    def comp_body(g, off):
        l = g // 8
        oc = (g % 8) * 16
        r16 = items_v[l, pl.ds(oc, 16)] - item0
        mask = (r16 >= 0) & (r16 < _IPS)
        c16 = lax.iota(jnp.int32, 16) + oc
        plsc.store_compressed(items_c.at[pl.ds(off, 16)], r16, mask=mask)
        plsc.store_compressed(cols_c.at[pl.ds(off, 16)], c16, mask=mask)
        plsc.store_compressed(vals_c.at[pl.ds(off, 16)],
                              vals_v[l, pl.ds(oc, 16)], mask=mask)
        return off + jnp.sum(mask.astype(jnp.int32))

    off = lax.fori_loop(0, _NG, comp_body, jnp.int32(0))
    items_c[pl.ds(off, 16)] = jnp.full((16,), -1, jnp.int32)
    ng = (off + 15) // 16

    def pass_fused(buf, prev_base, cur_base, cur_width):
        def g_body(g, _):
            o = g * 16
            r16 = items_c[pl.ds(o, 16)]
            c16 = cols_c[pl.ds(o, 16)]
            v16 = vals_c[pl.ds(o, 16)]
            rp = r16 - prev_base
            mp = (rp >= 0) & (rp < _T)
            rp = jnp.where(mp, rp, 0)
            plsc.store_scatter(buf, [rp, c16],
                               jnp.zeros((16,), jnp.float32), mask=mp)
            rc = r16 - cur_base
            mc = (rc >= 0) & (rc < cur_width)
            rc = jnp.where(mc, rc, 0)
            plsc.store_scatter(buf, [rc, c16], v16, mask=mc)
            return 0
        lax.fori_loop(0, ng, g_body, 0)

    def dst(k):
        row = pl.multiple_of(item0 + k * _T, 8)
        return out_hbm.at[pl.ds(row, _T), pl.ds(col0, 128)]

    pass_fused(buf_a, _NEG, 0 * _T, _T)
    pltpu.async_copy(buf_a, dst(0), sem_a)
    pass_fused(buf_b, _NEG, 1 * _T, _T)
    pltpu.async_copy(buf_b, dst(1), sem_b)

    def pair(m, _):
        for par, buf, sem in ((0, buf_a, sem_a), (1, buf_b, sem_b)):
            k = 2 * m + par
            pltpu.make_async_copy(buf, dst(k - 2), sem).wait()
            pass_fused(buf, (k - 2) * _T, k * _T, _T)
            pltpu.async_copy(buf, dst(k), sem)
        return 0

    lax.fori_loop(1, (_KF - 1) // 2, pair, 0)

    pltpu.make_async_copy(buf_a, dst(_KF - 3), sem_a).wait()
    pass_fused(buf_a, (_KF - 3) * _T, (_KF - 1) * _T, _T)
    pltpu.async_copy(buf_a, dst(_KF - 1), sem_a)
    pltpu.make_async_copy(buf_b, dst(_KF - 2), sem_b).wait()
    tail_base = _KF * _T
    pass_fused(buf_b, (_KF - 2) * _T, tail_base, _TAIL)
    tail_row = pl.multiple_of(item0 + tail_base, 8)
    tail_dst = out_hbm.at[pl.ds(tail_row, _TAIL), pl.ds(col0, 128)]
    pltpu.async_copy(buf_b.at[pl.ds(0, _TAIL), :], tail_dst, sem_b)
    pltpu.make_async_copy(buf_a, dst(_KF - 1), sem_a).wait()
    pltpu.make_async_copy(buf_b.at[pl.ds(0, _TAIL), :], tail_dst, sem_b).wait()


@functools.lru_cache(maxsize=1)
def _make_sc_scatter():
    return pl.kernel(
        _sc_scatter_body,
        out_type=jax.ShapeDtypeStruct((_NUM_ITEM, _B), jnp.float32),
        mesh=plsc.VectorSubcoreMesh(core_axis_name="c", subcore_axis_name="s"),
        compiler_params=pltpu.CompilerParams(needs_layout_passes=False),
        scratch_types=[
            pltpu.VMEM((_L, 128), jnp.int32),
            pltpu.VMEM((_L, 128), jnp.float32),
            pltpu.VMEM((_SPS + 16,), jnp.int32),
            pltpu.VMEM((_SPS + 16,), jnp.int32),
            pltpu.VMEM((_SPS + 16,), jnp.float32),
            pltpu.VMEM((_T, 128), jnp.float32),
            pltpu.VMEM((_T, 128), jnp.float32),
            pltpu.SemaphoreType.DMA,
            pltpu.SemaphoreType.DMA,
            pltpu.SemaphoreType.DMA,
        ],
    )


def kernel(all_memory, last_memory, seq_item, Wr, Ur, Vr_w, Vr_b):
    seq = seq_item.astype(jnp.int32)
    vals_t, items_t = _tc_probs(
        all_memory.transpose(1, 2, 0), last_memory.T, seq.T,
        Wr, Ur, Vr_w, Vr_b)
    zeros = jnp.zeros((_T, 128), jnp.float32)
    out_t = _make_sc_scatter()(items_t, vals_t, zeros)
    return out_t.T
